# baseline (speedup 1.0000x reference)
import functools

import jax
import jax.numpy as jnp
from jax import lax
from jax.experimental import pallas as pl
from jax.experimental.pallas import tpu as pltpu
from jax.experimental.pallas import tpu_sc as plsc

_D = 64
_B = 16384
_NC = 2
_NS = 16
_NW = _NC * _NS
_BPW = _B // _NW
_CHUNK = 128
_NCHUNK = _BPW // _CHUNK
_FPC = _CHUNK // 2
_GRP = _CHUNK // 16
_SCALE = 1.0 / float(_B * _D)


def _sc_body(fpair_hbm, labels_hbm, pairs_hbm, out_hbm,
             lab_v, idx_v, rows0, rows1, feat0, feat1, part_v, shared_v,
             gath_v, sem0, sem1):
    cid = lax.axis_index("c")
    sid = lax.axis_index("s")
    wid = sid * _NC + cid
    base = wid * _BPW

    rows_bufs = (rows0, rows1)
    feat_bufs = (feat0, feat1)
    sems = (sem0, sem1)

    pltpu.sync_copy(labels_hbm.at[pl.ds(base, _BPW)], lab_v)

    def idx_body(i, _):
        lab16 = lab_v[pl.ds(i * 16, 16)]
        idx_v[pl.ds(i * 16, 16)] = lax.shift_right_logical(lab16, 1)
        return 0

    lax.fori_loop(0, _BPW // 16, idx_body, 0)

    def start_chunk(k):
        b = k % 2
        g = pltpu.async_copy(
            pairs_hbm.at[idx_v.at[pl.ds(k * _CHUNK, _CHUNK)]],
            rows_bufs[b], sems[b])
        f = pltpu.async_copy(
            fpair_hbm.at[pl.ds(wid * (_BPW // 2) + k * _FPC, _FPC)],
            feat_bufs[b], sems[b])
        return g, f

    def compute_chunk(k, acc):
        b = k % 2
        rows_v = rows_bufs[b]
        feat_v = feat_bufs[b]
        k_base = k * _CHUNK

        def group_body(g, acc):
            for i in range(16):
                lab_splat = plsc.load_gather(
                    lab_v, [jnp.full((16,), k_base + g * 16 + i, jnp.int32)])
                p = jnp.bitwise_and(lab_splat, 1).astype(jnp.float32)
                half = i % 2
                q = g * 8 + i // 2
                s = g * 16 + i
                for c in range(4):
                    f = feat_v[q, pl.ds(half * 64 + c * 16, 16)]
                    left = rows_v[s, pl.ds(c * 16, 16)]
                    right = rows_v[s, pl.ds(64 + c * 16, 16)]
                    chosen = left + p * (right - left)
                    d = f - chosen
                    acc = acc + d * d
            return acc

        return lax.fori_loop(0, _GRP, group_body, acc)

    acc = jnp.zeros((16,), jnp.float32)
    pending = start_chunk(0)
    for k in range(_NCHUNK):
        for c in pending:
            c.wait()
        if k + 1 < _NCHUNK:
            pending = start_chunk(k + 1)
        acc = compute_chunk(k, acc)

    part_v[...] = acc
    pltpu.sync_copy(part_v, shared_v.at[pl.ds(sid * 16, 16)])
    plsc.subcore_barrier()

    @pl.when(sid == 0)
    def _():
        pltpu.sync_copy(shared_v, gath_v)
        tot = gath_v[pl.ds(0, 16)]
        for s in range(1, _NS):
            tot = tot + gath_v[pl.ds(s * 16, 16)]
        part_v[...] = tot * _SCALE
        pltpu.sync_copy(part_v, out_hbm.at[cid])


_SCRATCH = [
    pltpu.VMEM((_BPW,), jnp.int32),
    pltpu.VMEM((_BPW,), jnp.int32),
    pltpu.VMEM((_CHUNK, 128), jnp.float32),
    pltpu.VMEM((_CHUNK, 128), jnp.float32),
    pltpu.VMEM((_FPC, 128), jnp.float32),
    pltpu.VMEM((_FPC, 128), jnp.float32),
    pltpu.VMEM((16,), jnp.float32),
    pltpu.VMEM_SHARED((_NS * 16,), jnp.float32),
    pltpu.VMEM((_NS * 16,), jnp.float32),
    pltpu.SemaphoreType.DMA,
    pltpu.SemaphoreType.DMA,
]


@functools.partial(
    pl.kernel,
    out_type=jax.ShapeDtypeStruct((_NC, 16), jnp.float32),
    mesh=plsc.VectorSubcoreMesh(core_axis_name="c", subcore_axis_name="s"),
    scratch_types=_SCRATCH,
    compiler_params=pltpu.CompilerParams(needs_layout_passes=False),
)
def _center_loss_sc(fpair_hbm, labels_hbm, pairs_hbm, out_hbm,
                    lab_v, idx_v, rows0, rows1, feat0, feat1, part_v,
                    shared_v, gath_v, sem0, sem1):
    _sc_body(fpair_hbm, labels_hbm, pairs_hbm, out_hbm,
             lab_v, idx_v, rows0, rows1, feat0, feat1, part_v, shared_v,
             gath_v, sem0, sem1)


@jax.jit
def kernel(features, labels, centers):
    pairs = centers.reshape(50000, 128)
    fpair = features.reshape(_B // 2, 128)
    partials = _center_loss_sc(fpair, labels, pairs)
    return jnp.sum(partials)


# speedup vs baseline: 1.0264x; 1.0264x over previous
import functools

import jax
import jax.numpy as jnp
from jax import lax
from jax.experimental import pallas as pl
from jax.experimental.pallas import tpu as pltpu
from jax.experimental.pallas import tpu_sc as plsc

_D = 64
_B = 16384
_NC = 2
_NS = 16
_NW = _NC * _NS
_BPW = _B // _NW
_CHUNK = 128
_NCHUNK = _BPW // _CHUNK
_FPC = _CHUNK // 2
_GRP = _CHUNK // 16
_SCALE = 1.0 / float(_B * _D)


def _sc_body(feat_hbm, labels_hbm, pairs_hbm, out_hbm,
             lab_v, idx_v, rows0, rows1, feat0, feat1, part_v, shared_v,
             gath_v, sem0, sem1):
    cid = lax.axis_index("c")
    sid = lax.axis_index("s")
    wid = sid * _NC + cid
    base = wid * _BPW

    rows_bufs = (rows0, rows1)
    feat_bufs = (feat0, feat1)
    sems = (sem0, sem1)

    pltpu.sync_copy(labels_hbm.at[pl.ds(base, _BPW)], lab_v)

    def idx_body(i, _):
        lab16 = lab_v[pl.ds(i * 16, 16)]
        idx_v[pl.ds(i * 16, 16)] = lax.shift_right_logical(lab16, 1)
        return 0

    lax.fori_loop(0, _BPW // 16, idx_body, 0)

    def start_chunk(k):
        b = k % 2
        g = pltpu.async_copy(
            pairs_hbm.at[idx_v.at[pl.ds(k * _CHUNK, _CHUNK)]],
            rows_bufs[b], sems[b])
        f = pltpu.async_copy(
            feat_hbm.at[pl.ds(base + k * _CHUNK, _CHUNK)],
            feat_bufs[b], sems[b])
        return g, f

    def compute_chunk(k, acc):
        b = k % 2
        rows_v = rows_bufs[b]
        feat_v = feat_bufs[b]
        k_base = k * _CHUNK

        def group_body(g, acc):
            for i in range(16):
                s = g * 16 + i
                lab_splat = plsc.load_gather(
                    lab_v, [jnp.full((16,), k_base + s, jnp.int32)])
                p = jnp.bitwise_and(lab_splat, 1).astype(jnp.float32)
                for c in range(4):
                    f = feat_v[s, pl.ds(c * 16, 16)]
                    left = rows_v[s, pl.ds(c * 16, 16)]
                    right = rows_v[s, pl.ds(64 + c * 16, 16)]
                    chosen = left + p * (right - left)
                    d = f - chosen
                    acc = acc + d * d
            return acc

        return lax.fori_loop(0, _GRP, group_body, acc)

    acc = jnp.zeros((16,), jnp.float32)
    pending = start_chunk(0)
    for k in range(_NCHUNK):
        for c in pending:
            c.wait()
        if k + 1 < _NCHUNK:
            pending = start_chunk(k + 1)
        acc = compute_chunk(k, acc)

    part_v[...] = acc
    pltpu.sync_copy(part_v, shared_v.at[pl.ds(sid * 16, 16)])
    plsc.subcore_barrier()

    @pl.when(sid == 0)
    def _():
        pltpu.sync_copy(shared_v, gath_v)
        tot = gath_v[pl.ds(0, 16)]
        for s in range(1, _NS):
            tot = tot + gath_v[pl.ds(s * 16, 16)]
        part_v[...] = tot * _SCALE
        pltpu.sync_copy(part_v, out_hbm.at[cid])


_SCRATCH = [
    pltpu.VMEM((_BPW,), jnp.int32),
    pltpu.VMEM((_BPW,), jnp.int32),
    pltpu.VMEM((_CHUNK, 128), jnp.float32),
    pltpu.VMEM((_CHUNK, 128), jnp.float32),
    pltpu.VMEM((_CHUNK, 64), jnp.float32),
    pltpu.VMEM((_CHUNK, 64), jnp.float32),
    pltpu.VMEM((16,), jnp.float32),
    pltpu.VMEM_SHARED((_NS * 16,), jnp.float32),
    pltpu.VMEM((_NS * 16,), jnp.float32),
    pltpu.SemaphoreType.DMA,
    pltpu.SemaphoreType.DMA,
]


@functools.partial(
    pl.kernel,
    out_type=jax.ShapeDtypeStruct((_NC, 16), jnp.float32),
    mesh=plsc.VectorSubcoreMesh(core_axis_name="c", subcore_axis_name="s"),
    scratch_types=_SCRATCH,
    compiler_params=pltpu.CompilerParams(needs_layout_passes=False),
)
def _center_loss_sc(feat_hbm, labels_hbm, pairs_hbm, out_hbm,
                    lab_v, idx_v, rows0, rows1, feat0, feat1, part_v,
                    shared_v, gath_v, sem0, sem1):
    _sc_body(feat_hbm, labels_hbm, pairs_hbm, out_hbm,
             lab_v, idx_v, rows0, rows1, feat0, feat1, part_v, shared_v,
             gath_v, sem0, sem1)


@jax.jit
def kernel(features, labels, centers):
    pairs = centers.reshape(50000, 128)
    partials = _center_loss_sc(features, labels, pairs)
    return jnp.sum(partials)


# speedup vs baseline: 1.0513x; 1.0242x over previous
import functools

import jax
import jax.numpy as jnp
from jax import lax
from jax.experimental import pallas as pl
from jax.experimental.pallas import tpu as pltpu
from jax.experimental.pallas import tpu_sc as plsc

_D = 64
_B = 16384
_NC = 2
_NS = 16
_NW = _NC * _NS
_BPW = _B // _NW
_CHUNK = 128
_NCHUNK = _BPW // _CHUNK
_GRP = _CHUNK // 16
_SCALE = 1.0 / float(_B * _D)


def _sc_body(feat_hbm, labels_hbm, centers_hbm, out_hbm,
             lab_v, rows0, rows1, feat0, feat1, part_v, shared_v,
             gath_v, sem0, sem1):
    cid = lax.axis_index("c")
    sid = lax.axis_index("s")
    wid = sid * _NC + cid
    base = wid * _BPW

    rows_bufs = (rows0, rows1)
    feat_bufs = (feat0, feat1)
    sems = (sem0, sem1)

    pltpu.sync_copy(labels_hbm.at[pl.ds(base, _BPW)], lab_v)

    def start_chunk(k):
        b = k % 2
        g = pltpu.async_copy(
            centers_hbm.at[lab_v.at[pl.ds(k * _CHUNK, _CHUNK)]],
            rows_bufs[b], sems[b])
        f = pltpu.async_copy(
            feat_hbm.at[pl.ds(base + k * _CHUNK, _CHUNK)],
            feat_bufs[b], sems[b])
        return g, f

    def compute_chunk(k, acc):
        b = k % 2
        rows_v = rows_bufs[b]
        feat_v = feat_bufs[b]

        def group_body(g, acc):
            for i in range(16):
                s = g * 16 + i
                for c in range(4):
                    f = feat_v[s, pl.ds(c * 16, 16)]
                    t = rows_v[s, pl.ds(c * 16, 16)]
                    d = f - t
                    acc = acc + d * d
            return acc

        return lax.fori_loop(0, _GRP, group_body, acc)

    acc = jnp.zeros((16,), jnp.float32)
    pending = start_chunk(0)
    for k in range(_NCHUNK):
        for c in pending:
            c.wait()
        if k + 1 < _NCHUNK:
            pending = start_chunk(k + 1)
        acc = compute_chunk(k, acc)

    part_v[...] = acc
    pltpu.sync_copy(part_v, shared_v.at[pl.ds(sid * 16, 16)])
    plsc.subcore_barrier()

    @pl.when(sid == 0)
    def _():
        pltpu.sync_copy(shared_v, gath_v)
        tot = gath_v[pl.ds(0, 16)]
        for s in range(1, _NS):
            tot = tot + gath_v[pl.ds(s * 16, 16)]
        part_v[...] = tot * _SCALE
        pltpu.sync_copy(part_v, out_hbm.at[cid])


_SCRATCH = [
    pltpu.VMEM((_BPW,), jnp.int32),
    pltpu.VMEM((_CHUNK, _D), jnp.float32),
    pltpu.VMEM((_CHUNK, _D), jnp.float32),
    pltpu.VMEM((_CHUNK, _D), jnp.float32),
    pltpu.VMEM((_CHUNK, _D), jnp.float32),
    pltpu.VMEM((16,), jnp.float32),
    pltpu.VMEM_SHARED((_NS * 16,), jnp.float32),
    pltpu.VMEM((_NS * 16,), jnp.float32),
    pltpu.SemaphoreType.DMA,
    pltpu.SemaphoreType.DMA,
]


@functools.partial(
    pl.kernel,
    out_type=jax.ShapeDtypeStruct((_NC, 16), jnp.float32),
    mesh=plsc.VectorSubcoreMesh(core_axis_name="c", subcore_axis_name="s"),
    scratch_types=_SCRATCH,
    compiler_params=pltpu.CompilerParams(use_tc_tiling_on_sc=False),
)
def _center_loss_sc(feat_hbm, labels_hbm, centers_hbm, out_hbm,
                    lab_v, rows0, rows1, feat0, feat1, part_v,
                    shared_v, gath_v, sem0, sem1):
    _sc_body(feat_hbm, labels_hbm, centers_hbm, out_hbm,
             lab_v, rows0, rows1, feat0, feat1, part_v, shared_v,
             gath_v, sem0, sem1)


@jax.jit
def kernel(features, labels, centers):
    partials = _center_loss_sc(features, labels, centers)
    return jnp.sum(partials)
